# baseline (speedup 1.0000x reference)
import jax
import jax.numpy as jnp
from jax import lax
from jax.experimental import pallas as pl
from jax.experimental.pallas import tpu as pltpu
from jax.experimental.pallas import tpu_sc as plsc

NC = 2
NS = 16
NW = NC * NS
CB = 128
LANES = 16


def kernel(inputs, embedding):
    batch, hist = inputs.shape
    vocab, features = embedding.shape
    assert batch % (NW * CB) == 0
    bw = batch // NW
    nchunk = bw // CB
    steps = hist * nchunk
    assert steps % 2 == 0
    assert vocab % 2 == 0 and 2 * features == 128
    assert nchunk == 4

    idx_t = inputs.T
    table2 = embedding.reshape(vocab // 2, 2 * features)

    def body(table_hbm, idx_hbm, out_hbm, idx_v, qv, rows0, rows1,
             ob0, ob1, sg0, sg1, so0, so1):
        rows = (rows0, rows1)
        outb = (ob0, ob1)
        sem_g = (sg0, sg1)
        sem_o = (so0, so1)
        wid = lax.axis_index("s") * NC + lax.axis_index("c")
        base = wid * bw

        pltpu.sync_copy(idx_hbm.at[:, pl.ds(base, bw)], idx_v)

        @pl.loop(0, hist)
        def _(h):
            @pl.loop(0, bw // LANES)
            def _(s):
                qv[h, pl.ds(s * LANES, LANES)] = (
                    idx_v[h, pl.ds(s * LANES, LANES)] >> 1)

        def fire_gather(t, buf, sem):
            h = t >> 2
            c = (t & (nchunk - 1)) * CB
            pltpu.async_copy(table_hbm.at[qv.at[h, pl.ds(c, CB)]], buf, sem)

        def wait_gather(t, buf, sem):
            h = t >> 2
            c = (t & (nchunk - 1)) * CB
            pltpu.make_async_copy(table_hbm.at[qv.at[h, pl.ds(c, CB)]],
                                  buf, sem).wait()

        def process(t, buf, ob):
            h = t >> 2
            c = (t & (nchunk - 1)) * CB
            for s in range(CB // LANES):
                raw = idx_v[h, pl.ds(c + s * LANES, LANES)]
                pofs = (raw & 1) * features
                bidx = lax.iota(jnp.int32, LANES) + (s * LANES)
                for f in range(features):
                    ob[f, pl.ds(s * LANES, LANES)] = plsc.load_gather(
                        buf, [bidx, pofs + f])

        def fire_out(t, ob, sem):
            h = t >> 2
            c = (t & (nchunk - 1)) * CB
            pltpu.async_copy(ob, out_hbm.at[h, :, pl.ds(base + c, CB)], sem)

        def drain_out(ob, sem):
            pltpu.make_async_copy(ob, out_hbm.at[0, :, pl.ds(base, CB)],
                                  sem).wait()

        fire_gather(0, rows[0], sem_g[0])

        @pl.loop(0, steps, step=2)
        def _(t0):
            for b in range(2):
                t = t0 + b
                nb = 1 - b

                @pl.when(t + 1 < steps)
                def _():
                    fire_gather(t + 1, rows[nb], sem_g[nb])

                wait_gather(t, rows[b], sem_g[b])

                @pl.when(t >= 2)
                def _():
                    drain_out(outb[b], sem_o[b])

                process(t, rows[b], outb[b])
                fire_out(t, outb[b], sem_o[b])

        drain_out(outb[0], sem_o[0])
        drain_out(outb[1], sem_o[1])

    res = pl.kernel(
        body,
        out_type=jax.ShapeDtypeStruct((hist, features, batch), jnp.float32),
        mesh=plsc.VectorSubcoreMesh(core_axis_name="c", subcore_axis_name="s"),
        scratch_types=[
            pltpu.VMEM((hist, bw), jnp.int32),
            pltpu.VMEM((hist, bw), jnp.int32),
            pltpu.VMEM((CB, 2 * features), jnp.float32),
            pltpu.VMEM((CB, 2 * features), jnp.float32),
            pltpu.VMEM((features, CB), jnp.float32),
            pltpu.VMEM((features, CB), jnp.float32),
            pltpu.SemaphoreType.DMA,
            pltpu.SemaphoreType.DMA,
            pltpu.SemaphoreType.DMA,
            pltpu.SemaphoreType.DMA,
        ],
        compiler_params=pltpu.CompilerParams(use_tc_tiling_on_sc=True,
                                             needs_layout_passes=False),
    )(table2, idx_t)
    return jnp.transpose(res, (2, 0, 1))


# speedup vs baseline: 1.2720x; 1.2720x over previous
import jax
import jax.numpy as jnp
from jax import lax
from jax.experimental import pallas as pl
from jax.experimental.pallas import tpu as pltpu
from jax.experimental.pallas import tpu_sc as plsc

NC = 2
NS = 16
NW = NC * NS
CB = 128
LANES = 16


def kernel(inputs, embedding):
    batch, hist = inputs.shape
    vocab, features = embedding.shape
    assert batch % (NW * CB) == 0
    bw = batch // NW
    nchunk = bw // CB
    steps = hist * nchunk
    assert steps % 2 == 0
    assert vocab % 2 == 0 and 2 * features == 128
    assert nchunk == 4

    idx_t = inputs.T
    table2 = jnp.pad(embedding.T, ((0, 2 * features - features), (0, 0))).T

    def body(table_hbm, idx_hbm, out_hbm, idx_v, rows0, rows1,
             ob0, ob1, sg0, sg1, so0, so1):
        rows = (rows0, rows1)
        outb = (ob0, ob1)
        sem_g = (sg0, sg1)
        sem_o = (so0, so1)
        wid = lax.axis_index("s") * NC + lax.axis_index("c")
        base = wid * bw

        pltpu.sync_copy(idx_hbm.at[:, pl.ds(base, bw)], idx_v)

        def fire_gather(t, buf, sem):
            h = t >> 2
            c = (t & (nchunk - 1)) * CB
            pltpu.async_copy(table_hbm.at[idx_v.at[h, pl.ds(c, CB)]],
                             buf, sem)

        def wait_gather(t, buf, sem):
            h = t >> 2
            c = (t & (nchunk - 1)) * CB
            pltpu.make_async_copy(table_hbm.at[idx_v.at[h, pl.ds(c, CB)]],
                                  buf, sem).wait()

        def process(t, buf, ob):
            fvecs = [lax.iota(jnp.int32, LANES) + (cc * LANES)
                     for cc in range(features // LANES)]

            @pl.loop(0, CB)
            def _(b):
                bvec = jnp.full((LANES,), 0, jnp.int32) + b
                for cc in range(features // LANES):
                    x = buf[b, pl.ds(cc * LANES, LANES)]
                    plsc.store_scatter(ob, [fvecs[cc], bvec], x)

        def fire_out(t, ob, sem):
            h = t >> 2
            c = (t & (nchunk - 1)) * CB
            pltpu.async_copy(ob, out_hbm.at[h, :, pl.ds(base + c, CB)], sem)

        def drain_out(ob, sem):
            pltpu.make_async_copy(ob, out_hbm.at[0, :, pl.ds(base, CB)],
                                  sem).wait()

        fire_gather(0, rows[0], sem_g[0])

        @pl.loop(0, steps, step=2)
        def _(t0):
            for b in range(2):
                t = t0 + b
                nb = 1 - b

                @pl.when(t + 1 < steps)
                def _():
                    fire_gather(t + 1, rows[nb], sem_g[nb])

                wait_gather(t, rows[b], sem_g[b])

                @pl.when(t >= 2)
                def _():
                    drain_out(outb[b], sem_o[b])

                process(t, rows[b], outb[b])
                fire_out(t, outb[b], sem_o[b])

        drain_out(outb[0], sem_o[0])
        drain_out(outb[1], sem_o[1])

    res = pl.kernel(
        body,
        out_type=jax.ShapeDtypeStruct((hist, features, batch), jnp.float32),
        mesh=plsc.VectorSubcoreMesh(core_axis_name="c", subcore_axis_name="s"),
        scratch_types=[
            pltpu.VMEM((hist, bw), jnp.int32),
            pltpu.VMEM((CB, 2 * features), jnp.float32),
            pltpu.VMEM((CB, 2 * features), jnp.float32),
            pltpu.VMEM((features, CB), jnp.float32),
            pltpu.VMEM((features, CB), jnp.float32),
            pltpu.SemaphoreType.DMA,
            pltpu.SemaphoreType.DMA,
            pltpu.SemaphoreType.DMA,
            pltpu.SemaphoreType.DMA,
        ],
        compiler_params=pltpu.CompilerParams(use_tc_tiling_on_sc=True,
                                             needs_layout_passes=False),
    )(table2, idx_t)
    return jnp.transpose(res, (2, 0, 1))


# speedup vs baseline: 1.7013x; 1.3376x over previous
import jax
import jax.numpy as jnp
from jax import lax
from jax.experimental import pallas as pl
from jax.experimental.pallas import tpu as pltpu
from jax.experimental.pallas import tpu_sc as plsc

NC = 2
NS = 16
NW = NC * NS
K = 4
LANES = 16


def kernel(inputs, embedding):
    batch, hist = inputs.shape
    features = embedding.shape[1]
    assert batch % NW == 0
    rows_per_w = batch // NW
    assert rows_per_w % (2 * K) == 0
    pairs = rows_per_w // (2 * K)
    assert features % LANES == 0
    fblocks = features // LANES

    def body(table_hbm, idx_hbm, out_hbm, idx0, idx1, rows0, rows1, sel,
             si0, si1, sg0, sg1, so):
        idxs = (idx0, idx1)
        rows = (rows0, rows1)
        sem_i = (si0, si1)
        sem_g = (sg0, sg1)
        wid = lax.axis_index("s") * NC + lax.axis_index("c")
        base = wid * rows_per_w

        def fire_idx(h, buf, sem):
            pltpu.async_copy(idx_hbm.at[pl.ds(base + h * K, K)], buf, sem)

        def wait_idx(h, buf, sem):
            pltpu.make_async_copy(idx_hbm.at[pl.ds(base + h * K, K)],
                                  buf, sem).wait()

        def fire_gathers(buf, idx_v, sem):
            for j in range(K):
                pltpu.async_copy(table_hbm.at[idx_v.at[j]], buf.at[j], sem)

        def wait_gathers(buf, idx_v, sem):
            for j in range(K):
                pltpu.make_async_copy(table_hbm.at[idx_v.at[j]],
                                      buf.at[j], sem).wait()

        def compact(buf, half):
            for j in range(K):
                @pl.loop(0, hist)
                def _(r):
                    for c in range(fblocks):
                        sel[half * K + j, r, pl.ds(c * LANES, LANES)] = (
                            buf[j, r, pl.ds(c * LANES, LANES)])

        def fire_out(p):
            pltpu.async_copy(sel, out_hbm.at[pl.ds(base + p * 2 * K, 2 * K)],
                             so)

        def drain_out():
            pltpu.make_async_copy(sel, out_hbm.at[pl.ds(base, 2 * K)],
                                  so).wait()

        fire_idx(0, idxs[0], sem_i[0])
        fire_idx(1, idxs[1], sem_i[1])
        wait_idx(0, idxs[0], sem_i[0])
        fire_gathers(rows[0], idxs[0], sem_g[0])

        @pl.loop(0, pairs)
        def _(p):
            for b in range(2):
                h = p * 2 + b
                nb = 1 - b
                @pl.when(h + 1 < 2 * pairs)
                def _():
                    wait_idx(h + 1, idxs[nb], sem_i[nb])
                    fire_gathers(rows[nb], idxs[nb], sem_g[nb])
                wait_gathers(rows[b], idxs[b], sem_g[b])
                @pl.when(h + 2 < 2 * pairs)
                def _():
                    fire_idx(h + 2, idxs[b], sem_i[b])
                if b == 0:
                    @pl.when(p > 0)
                    def _():
                        drain_out()
                compact(rows[b], b)
            fire_out(p)

        drain_out()

    padded_table = jnp.pad(embedding.T, ((0, 128 - features), (0, 0))).T
    return pl.kernel(
        body,
        out_type=jax.ShapeDtypeStruct((batch, hist, features), jnp.float32),
        mesh=plsc.VectorSubcoreMesh(core_axis_name="c", subcore_axis_name="s"),
        scratch_types=[
            pltpu.VMEM((K, hist), jnp.int32),
            pltpu.VMEM((K, hist), jnp.int32),
            pltpu.VMEM((K, hist, 128), jnp.float32),
            pltpu.VMEM((K, hist, 128), jnp.float32),
            pltpu.VMEM((2 * K, hist, features), jnp.float32),
            pltpu.SemaphoreType.DMA,
            pltpu.SemaphoreType.DMA,
            pltpu.SemaphoreType.DMA,
            pltpu.SemaphoreType.DMA,
            pltpu.SemaphoreType.DMA,
        ],
        compiler_params=pltpu.CompilerParams(use_tc_tiling_on_sc=True),
    )(padded_table, inputs)
